# baseline (speedup 1.0000x reference)
import functools

import jax
import jax.numpy as jnp
from jax import lax
from jax.experimental import pallas as pl
from jax.experimental.pallas import tpu as pltpu
from jax.experimental.pallas import tpu_sc as plsc

D_MODEL = 64
SCALE = 8.0

_info = plsc.get_sparse_core_info()
_NC = _info.num_cores
_NS = _info.num_subcores
_NW = _NC * _NS
_C = 128


def _embed_sc(idx2d, table):
    n_rows_total = idx2d.shape[0]
    rows_per_w = n_rows_total // _NW
    total = n_rows_total * _C

    mesh = plsc.VectorSubcoreMesh(core_axis_name="c", subcore_axis_name="s")

    @functools.partial(
        pl.kernel,
        mesh=mesh,
        compiler_params=pltpu.CompilerParams(use_tc_tiling_on_sc=False),
        out_type=jax.ShapeDtypeStruct((total, D_MODEL), jnp.float32),
        scratch_types=[
            pltpu.VMEM((rows_per_w, _C), jnp.int32),
            pltpu.VMEM((_C, D_MODEL), jnp.float32),
            pltpu.SemaphoreType.DMA,
        ],
    )
    def k(idx_hbm, table_hbm, out_hbm, idx_v, rows_v, sem):
        wid = lax.axis_index("s") * _NC + lax.axis_index("c")
        ibase = wid * rows_per_w
        pltpu.sync_copy(idx_hbm.at[pl.ds(ibase, rows_per_w)], idx_v)

        def chunk_body(j, _):
            pltpu.async_copy(table_hbm.at[idx_v.at[j]], rows_v, sem).wait()

            def mul_body(i, _):
                for d in range(D_MODEL // 16):
                    s = pl.ds(d * 16, 16)
                    rows_v[i, s] = rows_v[i, s] * SCALE
                return ()

            lax.fori_loop(0, _C, mul_body, ())
            pltpu.sync_copy(rows_v, out_hbm.at[pl.ds((ibase + j) * _C, _C)])
            return ()

        lax.fori_loop(0, rows_per_w, chunk_body, ())

    return k(idx2d, table)


def kernel(token_ids, embedding_weights):
    s0, s1 = token_ids.shape
    idx = token_ids.astype(jnp.int32).reshape(-1, _C)
    out = _embed_sc(idx, embedding_weights)
    return out.reshape(s0, s1, D_MODEL)


# speedup vs baseline: 1.2493x; 1.2493x over previous
import functools

import jax
import jax.numpy as jnp
from jax import lax
from jax.experimental import pallas as pl
from jax.experimental.pallas import tpu as pltpu
from jax.experimental.pallas import tpu_sc as plsc

D_MODEL = 64
SCALE = 8.0
ROW = 128

_info = plsc.get_sparse_core_info()
_NC = _info.num_cores
_NS = _info.num_subcores
_NW = _NC * _NS
_C = 128


def _embed_sc(idx2d, table_padded):
    n_rows_total = idx2d.shape[0]
    rows_per_w = n_rows_total // _NW
    total = n_rows_total * _C

    mesh = plsc.VectorSubcoreMesh(core_axis_name="c", subcore_axis_name="s")

    @functools.partial(
        pl.kernel,
        mesh=mesh,
        out_type=jax.ShapeDtypeStruct((total, ROW), jnp.float32),
        scratch_types=[
            pltpu.VMEM((rows_per_w, _C), jnp.int32),
            pltpu.VMEM((_C, ROW), jnp.float32),
            pltpu.VMEM((_C, ROW), jnp.float32),
            pltpu.SemaphoreType.DMA,
            pltpu.SemaphoreType.DMA,
            pltpu.SemaphoreType.DMA,
            pltpu.SemaphoreType.DMA,
        ],
    )
    def k(idx_hbm, table_hbm, out_hbm, idx_v, buf0, buf1, g0, g1, o0, o1):
        wid = lax.axis_index("s") * _NC + lax.axis_index("c")
        ibase = wid * rows_per_w
        pltpu.sync_copy(idx_hbm.at[pl.ds(ibase, rows_per_w)], idx_v)

        bufs = (buf0, buf1)
        gsems = (g0, g1)
        osems = (o0, o1)

        def scale_rows(buf):
            def mul_body(i, _):
                for d in range(D_MODEL // 16):
                    s = pl.ds(d * 16, 16)
                    buf[i, s] = buf[i, s] * SCALE
                return ()

            lax.fori_loop(0, _C, mul_body, ())

        def gather(j, b):
            pltpu.async_copy(table_hbm.at[idx_v.at[j]], bufs[b], gsems[b])

        def wait_gather(b):
            pltpu.make_async_copy(table_hbm.at[idx_v.at[0]], bufs[b],
                                  gsems[b]).wait()

        def put(j, b):
            pltpu.async_copy(bufs[b], out_hbm.at[pl.ds((ibase + j) * _C, _C)],
                             osems[b])

        def wait_put(b):
            pltpu.make_async_copy(bufs[b], out_hbm.at[pl.ds(ibase * _C, _C)],
                                  osems[b]).wait()

        gather(0, 0)
        wait_gather(0)
        scale_rows(buf0)
        gather(1, 1)
        put(0, 0)

        def step(jj, _):
            j = 1 + 2 * jj
            wait_gather(1)
            scale_rows(buf1)
            wait_put(0)
            gather(j + 1, 0)
            put(j, 1)
            wait_gather(0)
            scale_rows(buf0)
            wait_put(1)
            gather(j + 2, 1)
            put(j + 1, 0)
            return ()

        lax.fori_loop(0, (rows_per_w - 2) // 2, step, ())

        wait_gather(1)
        scale_rows(buf1)
        put(rows_per_w - 1, 1)
        wait_put(0)
        wait_put(1)

    return k(idx2d, table_padded)


def kernel(token_ids, embedding_weights):
    s0, s1 = token_ids.shape
    idx = token_ids.astype(jnp.int32).reshape(-1, _C)
    tab = jnp.pad(embedding_weights, ((0, 0), (0, ROW - D_MODEL)))
    out = _embed_sc(idx, tab)
    return out.reshape(s0, s1, ROW)[..., :D_MODEL]
